# baseline (speedup 1.0000x reference)
import functools

import jax
import jax.numpy as jnp
from jax import lax
from jax.experimental import pallas as pl
from jax.experimental.pallas import tpu as pltpu
from jax.experimental.pallas import tpu_sc as plsc

VOCAB = 1000000
EMBED = 64
OUT = 4
SEQ = 200
BATCH = 4096

NC = 2
NS = 16
NW = NC * NS
BPW = BATCH // NW
RING = 8
LEAD = 6
INV_SEQ = 1.0 / SEQ

_mesh = plsc.VectorSubcoreMesh(
    core_axis_name="c", subcore_axis_name="s", num_cores=NC, num_subcores=NS
)


@functools.partial(
    pl.kernel,
    out_type=jax.ShapeDtypeStruct((BATCH, EMBED), jnp.float32),
    mesh=_mesh,
    scratch_types=[
        pltpu.VMEM((SEQ, BPW), jnp.int32),
        pltpu.VMEM((RING, BPW, EMBED), jnp.float32),
        pltpu.VMEM((BPW, EMBED), jnp.float32),
        pltpu.VMEM((BPW,), jnp.int32),
        pltpu.VMEM_SHARED((BATCH // NC, EMBED), jnp.float32),
        pltpu.SemaphoreType.DMA((RING,)),
        pltpu.SemaphoreType.DMA((RING,)),
    ],
    compiler_params=pltpu.CompilerParams(use_tc_tiling_on_sc=False),
)
def _pooled_kernel(text_hbm, table_hbm, pooled_hbm, idx_v, rows_v, pooled_v,
                   sidx_v, pooled_sh, gsem, ssem):
    wid = lax.axis_index("s") * NC + lax.axis_index("c")
    bbase = wid * BPW
    tb = lax.axis_index("s") * BPW

    pltpu.sync_copy(text_hbm.at[:, pl.ds(bbase, BPW)], idx_v)

    iota16 = lax.broadcasted_iota(jnp.int32, (16,), 0)
    zeros = jnp.zeros((16,), jnp.float32)
    for g in range(BPW // 16):
        sidx_v[pl.ds(16 * g, 16)] = iota16 + (16 * g) + tb

    def zero_body(j, carry):
        for g in range(EMBED // 16):
            pooled_v[j, pl.ds(16 * g, 16)] = zeros
        return carry

    lax.fori_loop(0, BPW, zero_body, 0)
    pltpu.sync_copy(pooled_v, pooled_sh.at[pl.ds(tb, BPW)])

    def gather(s, k):
        pltpu.async_copy(table_hbm.at[idx_v.at[s]], rows_v.at[k], gsem.at[k])

    def gather_wait(k):
        pltpu.make_async_copy(
            table_hbm.at[idx_v.at[0]], rows_v.at[k], gsem.at[k]
        ).wait()

    def scatter_add(k):
        pltpu.async_copy(
            rows_v.at[k], pooled_sh.at[sidx_v], ssem.at[k], add=True
        )

    def scatter_wait(k):
        pltpu.make_async_copy(
            rows_v.at[k], pooled_sh.at[sidx_v], ssem.at[k]
        ).wait()

    for k in range(LEAD):
        gather(k, k)

    def body(i, carry):
        for k in range(RING):
            s = RING * i + k
            gather_wait(k)
            scatter_add(k)
            kp = (k + LEAD) % RING

            @pl.when(s + LEAD < SEQ)
            def _():
                @pl.when(s >= RING - LEAD)
                def _():
                    scatter_wait(kp)

                gather(s + LEAD, kp)

        return carry

    lax.fori_loop(0, SEQ // RING, body, 0)

    for k in range(RING):
        scatter_wait(k)

    pltpu.sync_copy(pooled_sh.at[pl.ds(tb, BPW)], pooled_hbm.at[pl.ds(bbase, BPW)])


def _mm_body(p_ref, w_ref, b_ref, o_ref):
    o_ref[...] = (
        jnp.dot(p_ref[...], w_ref[...] * INV_SEQ,
                preferred_element_type=jnp.float32)
        + b_ref[...]
    )


_mm = pl.pallas_call(
    _mm_body,
    out_shape=jax.ShapeDtypeStruct((BATCH, OUT), jnp.float32),
)


def kernel(text, emb_table, fc_w, fc_b):
    pooled = _pooled_kernel(text.astype(jnp.int32), emb_table)
    return _mm(pooled, fc_w, fc_b.reshape(1, OUT))


# speedup vs baseline: 1.1696x; 1.1696x over previous
import functools

import jax
import jax.numpy as jnp
from jax import lax
from jax.experimental import pallas as pl
from jax.experimental.pallas import tpu as pltpu
from jax.experimental.pallas import tpu_sc as plsc

VOCAB = 1000000
EMBED = 64
OUT = 4
OUTP = 16
SEQ = 200
BATCH = 4096

NC = 2
NS = 16
NW = NC * NS
BPW = BATCH // NW
RING = 8
LEAD = 6
INV_SEQ = 1.0 / SEQ

VB = 12800

_mesh = plsc.VectorSubcoreMesh(
    core_axis_name="c", subcore_axis_name="s", num_cores=NC, num_subcores=NS
)


def _proj_body(tcm_ref, w_ref, b_ref, o_ref):
    o_ref[...] = (
        lax.dot_general(
            tcm_ref[...], w_ref[...] * INV_SEQ,
            dimension_numbers=(((0,), (0,)), ((), ())),
            preferred_element_type=jnp.float32,
        )
        + b_ref[...] * INV_SEQ
    )


_proj = pl.pallas_call(
    _proj_body,
    grid=((VOCAB + VB - 1) // VB,),
    in_specs=[
        pl.BlockSpec((EMBED, VB), lambda i: (0, i)),
        pl.BlockSpec((EMBED, OUTP), lambda i: (0, 0)),
        pl.BlockSpec((1, OUTP), lambda i: (0, 0)),
    ],
    out_specs=pl.BlockSpec((VB, OUTP), lambda i: (i, 0)),
    out_shape=jax.ShapeDtypeStruct((VOCAB, OUTP), jnp.float32),
)


@functools.partial(
    pl.kernel,
    out_type=jax.ShapeDtypeStruct((BATCH, OUTP), jnp.float32),
    mesh=_mesh,
    scratch_types=[
        pltpu.VMEM((SEQ, BPW), jnp.int32),
        pltpu.VMEM((RING, BPW, OUTP), jnp.float32),
        pltpu.VMEM((BPW,), jnp.int32),
        pltpu.VMEM_SHARED((BATCH // NC, OUTP), jnp.float32),
        pltpu.SemaphoreType.DMA((RING,)),
        pltpu.SemaphoreType.DMA((RING,)),
    ],
    compiler_params=pltpu.CompilerParams(use_tc_tiling_on_sc=False),
)
def _pool_kernel(text_hbm, proj_hbm, zeros_hbm, out_hbm, idx_v, rows_v,
                 sidx_v, pooled_sh, gsem, ssem):
    wid = lax.axis_index("s") * NC + lax.axis_index("c")
    bbase = wid * BPW
    tb = lax.axis_index("s") * BPW

    pltpu.sync_copy(text_hbm.at[:, pl.ds(bbase, BPW)], idx_v)
    pltpu.sync_copy(zeros_hbm, pooled_sh.at[pl.ds(tb, BPW)])

    iota16 = lax.broadcasted_iota(jnp.int32, (16,), 0)
    for g in range(BPW // 16):
        sidx_v[pl.ds(16 * g, 16)] = iota16 + (16 * g) + tb

    def gather(s, k):
        pltpu.async_copy(proj_hbm.at[idx_v.at[s]], rows_v.at[k], gsem.at[k])

    def gather_wait(k):
        pltpu.make_async_copy(
            proj_hbm.at[idx_v.at[0]], rows_v.at[k], gsem.at[k]
        ).wait()

    def scatter_add(k):
        pltpu.async_copy(
            rows_v.at[k], pooled_sh.at[sidx_v], ssem.at[k], add=True
        )

    def scatter_wait(k):
        pltpu.make_async_copy(
            rows_v.at[k], pooled_sh.at[sidx_v], ssem.at[k]
        ).wait()

    for k in range(LEAD):
        gather(k, k)

    def body(i, carry):
        for k in range(RING):
            s = RING * i + k
            gather_wait(k)
            scatter_add(k)
            kp = (k + LEAD) % RING

            @pl.when(s + LEAD < SEQ)
            def _():
                @pl.when(s >= RING - LEAD)
                def _():
                    scatter_wait(kp)

                gather(s + LEAD, kp)

        return carry

    lax.fori_loop(0, SEQ // RING, body, 0)

    for k in range(RING):
        scatter_wait(k)

    pltpu.sync_copy(pooled_sh.at[pl.ds(tb, BPW)], out_hbm.at[pl.ds(bbase, BPW)])


def kernel(text, emb_table, fc_w, fc_b):
    w_pad = jnp.pad(fc_w, ((0, 0), (0, OUTP - OUT)))
    b_pad = jnp.pad(fc_b, (0, OUTP - OUT)).reshape(1, OUTP)
    projected = _proj(emb_table.T, w_pad, b_pad)
    zeros = jnp.zeros((BPW, OUTP), jnp.float32)
    return _pool_kernel(text.astype(jnp.int32), projected, zeros)[:, :OUT]
